# Initial kernel scaffold; baseline (speedup 1.0000x reference)
import jax
import jax.numpy as jnp
from jax.experimental import pallas as pl


def kernel(ctype, ccont, utype, num_unit, ctype_w, utype_w, ccont_w):
    raise NotImplementedError("write your pallas kernel here")


# speedup vs baseline: 1.1135x; 1.1135x over previous
import dataclasses
import functools

import jax
import jax.numpy as jnp
from jax import lax
from jax.experimental import pallas as pl
from jax.experimental.pallas import tpu as pltpu
from jax.experimental.pallas import tpu_sc as plsc

B, L, D = 16384, 200, 32
NC, NU = 100000, 100000

NCORES, NSUB, LANES = 2, 16, 16
NW = NCORES * NSUB
RPW = B // NW
CB = 8
NCHUNK = RPW // CB
IDX_N = CB * L
GATHER_WINDOWS = [(o, min(128, IDX_N - o)) for o in range(0, IDX_N, 128)]
ZERO_IDX = NC


def _sc_body(ct_hbm, cc_hbm, ut_hbm, nu_hbm, ctw_hbm, utw_hbm, out_hbm,
             ctidx, utidx, ccv, nuv, ctrows, utrows, outv, sem):
    wid = lax.axis_index("c") * NSUB + lax.axis_index("s")
    row0 = wid * RPW

    pltpu.sync_copy(nu_hbm.at[pl.ds(row0, RPW)], nuv)

    @pl.loop(0, NCHUNK)
    def _chunk(ci):
        hoff = row0 * L + ci * IDX_N
        pltpu.sync_copy(ct_hbm.at[pl.ds(hoff, IDX_N)], ctidx)
        pltpu.sync_copy(ut_hbm.at[pl.ds(hoff, IDX_N)], utidx)
        pltpu.sync_copy(cc_hbm.at[pl.ds(hoff, IDX_N)], ccv)

        @pl.loop(0, CB)
        def _doctor(b):
            nusplat = plsc.load_gather(
                nuv, [jnp.full((LANES,), 0, jnp.int32) + (ci * CB + b)])
            for g in range(13):
                l0 = 184 if g == 12 else g * 16
                off = b * L + l0
                lvec = lax.iota(jnp.int32, LANES) + l0
                cc16 = ccv[pl.ds(off, LANES)]
                cur = ctidx[pl.ds(off, LANES)]
                valid = (lvec < nusplat) & (cc16 == 0)
                ctidx[pl.ds(off, LANES)] = jnp.where(valid, cur, ZERO_IDX)

        copies = []
        for off, w in GATHER_WINDOWS:
            sl = pl.ds(off, w)
            copies.append(pltpu.async_copy(
                ctw_hbm.at[ctidx.at[sl]], ctrows.at[sl], sem))
            copies.append(pltpu.async_copy(
                utw_hbm.at[utidx.at[sl]], utrows.at[sl], sem))
        for cp in copies:
            cp.wait()

        @pl.loop(0, CB)
        def _accum(b):
            def lbody(l, accs):
                a0, a1 = accs
                off = b * L + l
                c0 = ctrows[off, pl.ds(0, LANES)]
                c1 = ctrows[off, pl.ds(LANES, LANES)]
                u0 = utrows[off, pl.ds(0, LANES)]
                u1 = utrows[off, pl.ds(LANES, LANES)]
                return (a0 + c0 * u0, a1 + c1 * u1)

            z = jnp.zeros((LANES,), jnp.float32)
            a0, a1 = lax.fori_loop(0, L, lbody, (z, z))
            outv[b, pl.ds(0, LANES)] = a0
            outv[b, pl.ds(LANES, LANES)] = a1

        pltpu.sync_copy(outv, out_hbm.at[pl.ds(row0 + ci * CB, CB)])


def kernel(ctype, ccont, utype, num_unit, ctype_w, utype_w, ccont_w):
    del ccont_w
    ct1 = ctype.reshape(B * L)
    ut1 = utype.reshape(B * L)
    cc1 = ccont.reshape(B * L)
    ctw = jnp.concatenate(
        [ctype_w, jnp.zeros((8, D), ctype_w.dtype)], axis=0)

    mesh = plsc.VectorSubcoreMesh(
        core_axis_name="c", subcore_axis_name="s",
        num_cores=NCORES, num_subcores=NSUB)
    cp = pltpu.CompilerParams(
        needs_layout_passes=False, use_tc_tiling_on_sc=False)
    run = pl.kernel(
        _sc_body,
        out_type=jax.ShapeDtypeStruct((B, D), jnp.float32),
        mesh=mesh,
        compiler_params=cp,
        scratch_types=[
            pltpu.VMEM((IDX_N,), jnp.int32),
            pltpu.VMEM((IDX_N,), jnp.int32),
            pltpu.VMEM((IDX_N,), jnp.int32),
            pltpu.VMEM((RPW,), jnp.int32),
            pltpu.VMEM((IDX_N, D), jnp.float32),
            pltpu.VMEM((IDX_N, D), jnp.float32),
            pltpu.VMEM((CB, D), jnp.float32),
            pltpu.SemaphoreType.DMA,
        ],
    )
    return run(ct1, cc1, ut1, num_unit, ctw, utype_w)


# speedup vs baseline: 36.0467x; 32.3729x over previous
import jax
import jax.numpy as jnp
from jax import lax
from jax.experimental import pallas as pl
from jax.experimental.pallas import tpu as pltpu
from jax.experimental.pallas import tpu_sc as plsc

B, L, D = 16384, 200, 32
NC, NU = 100000, 100000

NCORES, NSUB, LANES = 2, 16, 16
NW = NCORES * NSUB
RPW = B // NW
CB = 8
NCHUNK = RPW // CB
IDX_N = CB * L
WIN = 128
NWIN_MAX = -(-IDX_N // WIN)
CAP = NWIN_MAX * WIN


def _splat(v):
    return jnp.full((LANES,), 0, jnp.int32) + v


def _to_scalar(vec):
    return lax.reduce_max(vec, axes=(0,))


def _sc_body(ct_hbm, cc_hbm, ut_hbm, nu_hbm, ctw_hbm, utw_hbm, out_hbm,
             ctraw, utraw, ccv, nuv, ctc, utc, starts, ctrows, utrows,
             outv, sem):
    wid = lax.axis_index("c") * NSUB + lax.axis_index("s")
    row0 = wid * RPW
    lane = lax.iota(jnp.int32, LANES)
    lane0 = lane == 0

    pltpu.sync_copy(nu_hbm.at[pl.ds(row0, RPW)], nuv)

    @pl.loop(0, CAP // LANES)
    def _fill(i):
        z = jnp.zeros((LANES,), jnp.int32)
        ctc[pl.ds(i * LANES, LANES)] = z
        utc[pl.ds(i * LANES, LANES)] = z

    @pl.loop(0, NCHUNK)
    def _chunk(ci):
        hoff = row0 * L + ci * IDX_N
        pltpu.sync_copy(ct_hbm.at[pl.ds(hoff, IDX_N)], ctraw)
        pltpu.sync_copy(ut_hbm.at[pl.ds(hoff, IDX_N)], utraw)
        pltpu.sync_copy(cc_hbm.at[pl.ds(hoff, IDX_N)], ccv)

        def row_compact(b, off):
            plsc.store_scatter(starts, [_splat(b)], off, mask=lane0)
            nusplat = plsc.load_gather(nuv, [_splat(ci * CB + b)])
            for g in range(13):
                l0 = g * 16
                base = b * L + l0
                lvec = lane + l0
                cc16 = ccv[pl.ds(base, LANES)]
                valid = (lvec < nusplat) & (cc16 == 0)
                if g == 12:
                    valid = valid & (lane < 8)
                cs = plsc.cumsum(valid.astype(jnp.int32))
                dst = off + cs - 1
                plsc.store_scatter(ctc, [dst], ctraw[pl.ds(base, LANES)],
                                   mask=valid)
                plsc.store_scatter(utc, [dst], utraw[pl.ds(base, LANES)],
                                   mask=valid)
                off = off + plsc.all_reduce_population_count(valid)
            return off

        off = lax.fori_loop(0, CB, row_compact,
                            jnp.zeros((LANES,), jnp.int32))
        plsc.store_scatter(starts, [_splat(CB)], off, mask=lane0)
        total = _to_scalar(off)
        nwin = (total + (WIN - 1)) // WIN

        @pl.loop(0, nwin)
        def _fire(j):
            sl = pl.ds(j * WIN, WIN)
            pltpu.async_copy(ctw_hbm.at[ctc.at[sl]], ctrows.at[sl], sem)
            pltpu.async_copy(utw_hbm.at[utc.at[sl]], utrows.at[sl], sem)

        @pl.loop(0, nwin)
        def _drain(j):
            sl = pl.ds(j * WIN, WIN)
            pltpu.make_async_copy(ctw_hbm.at[ctc.at[sl]],
                                  ctrows.at[sl], sem).wait()
            pltpu.make_async_copy(utw_hbm.at[utc.at[sl]],
                                  utrows.at[sl], sem).wait()

        @pl.loop(0, CB)
        def _accum(b):
            s = _to_scalar(plsc.load_gather(starts, [_splat(b)]))
            e = _to_scalar(plsc.load_gather(starts, [_splat(b + 1)]))

            def lbody(l, accs):
                a0, a1 = accs
                c2 = ctrows[l, pl.ds(0, 2 * LANES)]
                u2 = utrows[l, pl.ds(0, 2 * LANES)]
                pa, pb = plsc.unpack(c2 * u2,
                                     format=plsc.PackFormat.INTERLEAVED)
                return (a0 + pa, a1 + pb)

            z = jnp.zeros((LANES,), jnp.float32)
            a0, a1 = lax.fori_loop(s, e, lbody, (z, z))
            outv[b, pl.ds(0, LANES)] = a0
            outv[b, pl.ds(LANES, LANES)] = a1

        pltpu.sync_copy(outv, out_hbm.at[pl.ds(row0 + ci * CB, CB)])


def kernel(ctype, ccont, utype, num_unit, ctype_w, utype_w, ccont_w):
    del ccont_w
    ct1 = ctype.reshape(B * L)
    ut1 = utype.reshape(B * L)
    cc1 = ccont.reshape(B * L)
    perm = jnp.arange(D).reshape(2, D // 2).T.reshape(D)
    ctw = ctype_w[:, perm].astype(jnp.bfloat16)
    utw = utype_w[:, perm].astype(jnp.bfloat16)

    mesh = plsc.VectorSubcoreMesh(
        core_axis_name="c", subcore_axis_name="s",
        num_cores=NCORES, num_subcores=NSUB)
    cp = pltpu.CompilerParams(
        needs_layout_passes=False, use_tc_tiling_on_sc=False)
    run = pl.kernel(
        _sc_body,
        out_type=jax.ShapeDtypeStruct((B, D), jnp.float32),
        mesh=mesh,
        compiler_params=cp,
        scratch_types=[
            pltpu.VMEM((IDX_N,), jnp.int32),
            pltpu.VMEM((IDX_N,), jnp.int32),
            pltpu.VMEM((IDX_N,), jnp.int32),
            pltpu.VMEM((RPW,), jnp.int32),
            pltpu.VMEM((CAP,), jnp.int32),
            pltpu.VMEM((CAP,), jnp.int32),
            pltpu.VMEM((16,), jnp.int32),
            pltpu.VMEM((CAP, D), jnp.bfloat16),
            pltpu.VMEM((CAP, D), jnp.bfloat16),
            pltpu.VMEM((CB, D), jnp.float32),
            pltpu.SemaphoreType.DMA,
        ],
    )
    return run(ct1, cc1, ut1, num_unit, ctw, utw)
